# baseline (speedup 1.0000x reference)
import jax
import jax.numpy as jnp
from jax.experimental import pallas as pl
from jax.experimental.pallas import tpu as pltpu

_B, _D1, _F = 4096, 256, 128
_CORES = 2
_C = 512
_NSTEP = (_B // _CORES) // _C


def _stream_kernel(x0_hbm, x1_hbm, x2_hbm, o_hbm,
                   x0_buf, x1_buf, x2_buf, o_buf, in_sem, out_sem):
    base = pl.program_id(0) * (_B // _CORES)

    def dma_in(slot, step):
        r = base + step * _C
        pltpu.make_async_copy(x0_hbm.at[pl.ds(r, _C)], x0_buf.at[slot], in_sem.at[slot, 0]).start()
        pltpu.make_async_copy(x1_hbm.at[pl.ds(r, _C)], x1_buf.at[slot], in_sem.at[slot, 1]).start()
        pltpu.make_async_copy(x2_hbm.at[pl.ds(r, _C)], x2_buf.at[slot], in_sem.at[slot, 2]).start()

    def wait_in(slot):
        pltpu.make_async_copy(x0_hbm.at[pl.ds(0, _C)], x0_buf.at[slot], in_sem.at[slot, 0]).wait()
        pltpu.make_async_copy(x1_hbm.at[pl.ds(0, _C)], x1_buf.at[slot], in_sem.at[slot, 1]).wait()
        pltpu.make_async_copy(x2_hbm.at[pl.ds(0, _C)], x2_buf.at[slot], in_sem.at[slot, 2]).wait()

    def dma_out(slot, step):
        r = base + step * _C
        pltpu.make_async_copy(o_buf.at[slot], o_hbm.at[pl.ds(r, _C)], out_sem.at[slot]).start()

    def wait_out(slot):
        pltpu.make_async_copy(o_buf.at[slot], o_hbm.at[pl.ds(0, _C)], out_sem.at[slot]).wait()

    dma_in(0, 0)

    def body(step, _):
        cur = jax.lax.rem(step, 2)
        nxt = jax.lax.rem(step + 1, 2)

        @pl.when(step + 1 < _NSTEP)
        def _():
            dma_in(nxt, step + 1)

        wait_in(cur)

        @pl.when(step >= 2)
        def _():
            wait_out(cur)

        x0 = x0_buf[cur]
        x1 = x1_buf[cur]
        mx = jnp.max(x1, axis=1, keepdims=True)
        mn = jnp.min(x1, axis=1, keepdims=True)
        o_buf[cur, :, :_D1] = jnp.maximum(x0 * mx, x0 * mn)
        o_buf[cur, :, _D1:] = x2_buf[cur]
        dma_out(cur, step)
        return ()

    jax.lax.fori_loop(0, _NSTEP, body, (), unroll=True)
    wait_out(jax.lax.rem(_NSTEP - 2, 2))
    wait_out(jax.lax.rem(_NSTEP - 1, 2))


def kernel(x0, x1, x2):
    return pl.pallas_call(
        _stream_kernel,
        grid=(_CORES,),
        in_specs=[
            pl.BlockSpec(memory_space=pl.ANY),
            pl.BlockSpec(memory_space=pl.ANY),
            pl.BlockSpec(memory_space=pl.ANY),
        ],
        out_specs=pl.BlockSpec(memory_space=pl.ANY),
        out_shape=jax.ShapeDtypeStruct((_B, _D1 + _F), x0.dtype),
        scratch_shapes=[
            pltpu.VMEM((2, _C, _D1), jnp.float32),
            pltpu.VMEM((2, _C, _D1), jnp.float32),
            pltpu.VMEM((2, _C, _F), jnp.float32),
            pltpu.VMEM((2, _C, _D1 + _F), jnp.float32),
            pltpu.SemaphoreType.DMA((2, 3)),
            pltpu.SemaphoreType.DMA((2,)),
        ],
        compiler_params=pltpu.CompilerParams(
            dimension_semantics=("parallel",),
        ),
    )(x0, x1, x2)


# speedup vs baseline: 1.2974x; 1.2974x over previous
import jax
import jax.numpy as jnp
from jax.experimental import pallas as pl
from jax.experimental.pallas import tpu as pltpu

_B, _D1, _F = 4096, 256, 128
_BLK = 4096


def _pairwise_max_kernel(x0_ref, x1_ref, x2_ref, out_ref):
    x0 = x0_ref[...]
    x1 = x1_ref[...]
    mx = jnp.max(x1, axis=1, keepdims=True)
    mn = jnp.min(x1, axis=1, keepdims=True)
    out_ref[:, :_D1] = jnp.maximum(x0 * mx, x0 * mn)
    out_ref[:, _D1:] = x2_ref[...]


def kernel(x0, x1, x2):
    B, D1 = x0.shape
    F = x2.shape[1]
    grid = (B // _BLK,)
    return pl.pallas_call(
        _pairwise_max_kernel,
        grid=grid,
        in_specs=[
            pl.BlockSpec((_BLK, D1), lambda i: (i, 0)),
            pl.BlockSpec((_BLK, x1.shape[1]), lambda i: (i, 0)),
            pl.BlockSpec((_BLK, F), lambda i: (i, 0)),
        ],
        out_specs=pl.BlockSpec((_BLK, D1 + F), lambda i: (i, 0)),
        out_shape=jax.ShapeDtypeStruct((B, D1 + F), x0.dtype),
        compiler_params=pltpu.CompilerParams(
            dimension_semantics=("parallel",),
        ),
    )(x0, x1, x2)


# speedup vs baseline: 1.6027x; 1.2353x over previous
import jax
import jax.numpy as jnp
from jax.experimental import pallas as pl
from jax.experimental.pallas import tpu as pltpu

_B, _D1, _F = 4096, 256, 128
_BLK = 2048


def _pairwise_max_kernel(x0_ref, x1_ref, x2_ref, out_ref):
    x0 = x0_ref[...]
    x1 = x1_ref[...]
    mx = jnp.max(x1, axis=1, keepdims=True)
    mn = jnp.min(x1, axis=1, keepdims=True)
    out_ref[:, :_D1] = jnp.maximum(x0 * mx, x0 * mn)
    out_ref[:, _D1:] = x2_ref[...]


def kernel(x0, x1, x2):
    B, D1 = x0.shape
    F = x2.shape[1]
    grid = (B // _BLK,)
    return pl.pallas_call(
        _pairwise_max_kernel,
        grid=grid,
        in_specs=[
            pl.BlockSpec((_BLK, D1), lambda i: (i, 0)),
            pl.BlockSpec((_BLK, x1.shape[1]), lambda i: (i, 0)),
            pl.BlockSpec((_BLK, F), lambda i: (i, 0)),
        ],
        out_specs=pl.BlockSpec((_BLK, D1 + F), lambda i: (i, 0)),
        out_shape=jax.ShapeDtypeStruct((B, D1 + F), x0.dtype),
        compiler_params=pltpu.CompilerParams(
            dimension_semantics=("arbitrary",),
        ),
    )(x0, x1, x2)
